# baseline (speedup 1.0000x reference)
import functools

import jax
import jax.numpy as jnp
from jax import lax
from jax.experimental import pallas as pl
from jax.experimental.pallas import tpu as pltpu
from jax.experimental.pallas import tpu_sc as plsc

MAX_DEG = 10
NC, NS = 2, 16
NW = NC * NS
CHUNK = 128


def _make_segsum(n_table, d_w, n_rows, chunk, npc, nblk, k):
    assert d_w % 16 == 0 and n_rows % (NS * 8) == 0
    assert npc % nblk == 0 and nblk % k == 0
    rows_per_tile = n_rows // NS
    mesh = plsc.VectorSubcoreMesh(core_axis_name="c", subcore_axis_name="s")

    @functools.partial(
        pl.kernel,
        mesh=mesh,
        out_type=jax.ShapeDtypeStruct((NC, n_rows, d_w), jnp.float32),
        scratch_types=(
            [pltpu.VMEM((nblk, chunk), jnp.int32),
             pltpu.VMEM((nblk, chunk), jnp.int32)]
            + [pltpu.VMEM((chunk, d_w), jnp.float32) for _ in range(k)]
            + [pltpu.VMEM_SHARED((n_rows, d_w), jnp.float32)]
            + [pltpu.SemaphoreType.DMA for _ in range(2 * k)]
        ),
        compiler_params=pltpu.CompilerParams(use_tc_tiling_on_sc=False),
    )
    def seg_kernel(table_hbm, src_hbm, dst_hbm, out_hbm, src_v, dst_v, *bufs):
        rows = bufs[:k]
        acc_sh = bufs[k]
        sg = bufs[k + 1:2 * k + 1]
        ss = bufs[2 * k + 1:]
        c = lax.axis_index("c")
        s = lax.axis_index("s")
        wid = s * NC + c

        def _zrow(r, _):
            for cc in range(d_w // 16):
                rows[0][r, pl.ds(cc * 16, 16)] = jnp.zeros((16,), jnp.float32)
            return 0
        lax.fori_loop(0, chunk, _zrow, 0)

        tile_base = pl.multiple_of(s * rows_per_tile, 8)
        done = 0
        while done < rows_per_tile:
            nr = min(chunk, rows_per_tile - done)
            pltpu.sync_copy(rows[0].at[pl.ds(0, nr)],
                            acc_sh.at[pl.ds(pl.multiple_of(tile_base + done, 8), nr)])
            done += nr
        plsc.subcore_barrier()

        def _block(b, _):
            pltpu.sync_copy(src_hbm.at[wid, pl.ds(b * nblk, nblk)], src_v)
            pltpu.sync_copy(dst_hbm.at[wid, pl.ds(b * nblk, nblk)], dst_v)

            def _group(g, _u):
                j0 = g * k
                gd = [pltpu.async_copy(table_hbm.at[src_v.at[j0 + j]], rows[j], sg[j])
                      for j in range(k)]
                sd = []
                for j in range(k):
                    gd[j].wait()
                    sd.append(pltpu.async_copy(rows[j], acc_sh.at[dst_v.at[j0 + j]],
                                               ss[j], add=True))
                for d_ in sd:
                    d_.wait()
                return 0
            lax.fori_loop(0, nblk // k, _group, 0)
            return 0
        lax.fori_loop(0, npc // nblk, _block, 0)
        plsc.subcore_barrier()

        done = 0
        while done < rows_per_tile:
            nr = min(chunk, rows_per_tile - done)
            lo = pl.multiple_of(tile_base + done, 8)
            pltpu.sync_copy(acc_sh.at[pl.ds(lo, nr)], rows[0].at[pl.ds(0, nr)])
            pltpu.sync_copy(rows[0].at[pl.ds(0, nr)], out_hbm.at[c, pl.ds(lo, nr)])
            done += nr

    return seg_kernel


def _deg_select(agg, idx, w_ref, b_ref):
    acc = jnp.zeros((agg.shape[0], w_ref.shape[2]), jnp.float32)
    for d in range(MAX_DEG):
        y = jnp.dot(agg, w_ref[d], preferred_element_type=jnp.float32) + b_ref[d][None, :]
        acc = jnp.where((idx == d)[:, None], y, acc)
    return acc


def _nf_layer_body(deg_col, h_ref, parts_ref, w_ref, b_ref, out_ref):
    din = h_ref.shape[1]
    agg = h_ref[...] + parts_ref[0, :, :din] + parts_ref[1, :, :din]
    degf = parts_ref[0, :, deg_col] + parts_ref[1, :, deg_col]
    idx = jnp.clip(degf.astype(jnp.int32), 1, MAX_DEG) - 1
    out_ref[...] = jnp.maximum(_deg_select(agg, idx, w_ref, b_ref), 0.0)


def _nf_layer_tc(h, parts, deg_col, w, b, blk):
    n, din = h.shape
    dout = w.shape[2]
    dw = parts.shape[2]
    grid = (n + blk - 1) // blk
    return pl.pallas_call(
        functools.partial(_nf_layer_body, deg_col),
        grid=(grid,),
        in_specs=[
            pl.BlockSpec((blk, din), lambda i: (i, 0)),
            pl.BlockSpec((2, blk, dw), lambda i: (0, i, 0)),
            pl.BlockSpec(w.shape, lambda i: (0, 0, 0)),
            pl.BlockSpec(b.shape, lambda i: (0, 0)),
        ],
        out_specs=pl.BlockSpec((blk, dout), lambda i: (i, 0)),
        out_shape=jax.ShapeDtypeStruct((n, dout), jnp.float32),
    )(h, parts, w, b)


def _readout_body(n_valid, nblocks, deg_col, h_ref, parts_ref, degp_ref, w_ref,
                  b_ref, wn_ref, bn_ref, wt_ref, bt_ref, out_ref, gsum, gmax):
    i = pl.program_id(0)
    blk = h_ref.shape[0]

    agg = h_ref[...] + parts_ref[0] + parts_ref[1]
    degf = degp_ref[0, :, deg_col] + degp_ref[1, :, deg_col]
    idx = jnp.clip(degf.astype(jnp.int32), 1, MAX_DEG) - 1
    h2 = jnp.maximum(_deg_select(agg, idx, w_ref, b_ref), 0.0)

    nf = jnp.dot(h2, wn_ref[...], preferred_element_type=jnp.float32) + bn_ref[...]
    row = i * blk + lax.broadcasted_iota(jnp.int32, (blk, 1), 0)
    valid = row < n_valid
    nsum = jnp.sum(jnp.where(valid, nf, 0.0), axis=0, keepdims=True)
    nmax = jnp.max(jnp.where(valid, nf, -jnp.inf), axis=0, keepdims=True)

    @pl.when(i == 0)
    def _():
        gsum[...] = jnp.zeros_like(gsum)
        gmax[...] = jnp.full_like(gmax, -jnp.inf)

    gsum[...] += nsum
    gmax[...] = jnp.maximum(gmax[...], nmax)

    @pl.when(i == nblocks - 1)
    def _():
        gf = jnp.maximum(jnp.concatenate([gsum[...], gmax[...]], axis=1), 0.0)
        out_ref[...] = jnp.dot(gf, wt_ref[...], preferred_element_type=jnp.float32) + bt_ref[...]


def _readout_tc(h, parts, degparts, deg_col, w, b, wn, bn, wt, bt, blk):
    n, din = h.shape
    p = wn.shape[1]
    out = wt.shape[1]
    dw = degparts.shape[2]
    grid = (n + blk - 1) // blk
    return pl.pallas_call(
        functools.partial(_readout_body, n, grid, deg_col),
        grid=(grid,),
        in_specs=[
            pl.BlockSpec((blk, din), lambda i: (i, 0)),
            pl.BlockSpec((2, blk, din), lambda i: (0, i, 0)),
            pl.BlockSpec((2, blk, dw), lambda i: (0, i, 0)),
            pl.BlockSpec(w.shape, lambda i: (0, 0, 0)),
            pl.BlockSpec(b.shape, lambda i: (0, 0)),
            pl.BlockSpec(wn.shape, lambda i: (0, 0)),
            pl.BlockSpec((1, p), lambda i: (0, 0)),
            pl.BlockSpec(wt.shape, lambda i: (0, 0)),
            pl.BlockSpec((1, out), lambda i: (0, 0)),
        ],
        out_specs=pl.BlockSpec((1, out), lambda i: (0, 0)),
        out_shape=jax.ShapeDtypeStruct((1, out), jnp.float32),
        scratch_shapes=[pltpu.VMEM((1, p), jnp.float32),
                        pltpu.VMEM((1, p), jnp.float32)],
    )(h, parts, degparts, w, b, wn, bn, wt, bt)


def kernel(x, edge_index, W1, b1, W2, b2, W_ntg, b_ntg, W_t, b_t):
    n, d = x.shape
    e = edge_index.shape[1]
    h1dim = W1.shape[2]

    d_aug = ((d + 1 + 15) // 16) * 16
    n_rows = ((n + 1 + 127) // 128) * 128

    ch1, nblk1, k1 = 32, 24, 6
    npc1 = -(-e // (NW * ch1))
    npc1 += (-npc1) % nblk1
    ch2, nblk2, k2 = 64, 24, 6
    npc2 = -(-e // (NW * ch2))
    npc2 += (-npc2) % nblk2
    e_pad = NW * ch1 * npc1
    assert e_pad == NW * ch2 * npc2

    src = edge_index[0].astype(jnp.int32)
    dst = edge_index[1].astype(jnp.int32)
    pad = e_pad - e
    pad_dst = n + jnp.arange(pad, dtype=jnp.int32) % (n_rows - n)
    src_p = jnp.concatenate([src, jnp.zeros((pad,), jnp.int32)])
    dst_p = jnp.concatenate([dst, pad_dst])

    x_aug = jnp.concatenate(
        [x, jnp.ones((n, 1), jnp.float32), jnp.zeros((n, d_aug - d - 1), jnp.float32)],
        axis=1)

    seg1 = _make_segsum(n, d_aug, n_rows, ch1, npc1, nblk1, k1)
    parts1 = seg1(x_aug, src_p.reshape(NW, npc1, ch1),
                  dst_p.reshape(NW, npc1, ch1))

    blk = 1024
    h1 = _nf_layer_tc(x, parts1, d, W1, b1, blk)

    seg2 = _make_segsum(n, h1dim, n_rows, ch2, npc2, nblk2, k2)
    parts2 = seg2(h1, src_p.reshape(NW, npc2, ch2),
                  dst_p.reshape(NW, npc2, ch2))

    return _readout_tc(h1, parts2, parts1, d, W2, b2, W_ntg,
                       b_ntg.reshape(1, -1), W_t, b_t.reshape(1, -1), blk)


# speedup vs baseline: 1.9352x; 1.9352x over previous
import functools

import jax
import jax.numpy as jnp
from jax import lax
from jax.experimental import pallas as pl
from jax.experimental.pallas import tpu as pltpu
from jax.experimental.pallas import tpu_sc as plsc

MAX_DEG = 10
NC, NS = 2, 16
NW = NC * NS
CHUNK = 128


def _make_segsum(n_table, d_w, n_rows, chunk, npc, nblk, stage):
    assert d_w % 16 == 0 and n_rows % (NS * 8) == 0
    assert npc % nblk == 0 and nblk % 2 == 0
    rows_per_tile = n_rows // NS
    mesh = plsc.VectorSubcoreMesh(core_axis_name="c", subcore_axis_name="s")

    scratch = [
        pltpu.VMEM((nblk, chunk), jnp.int32),
        pltpu.VMEM((nblk, chunk), jnp.int32),
        pltpu.VMEM((chunk, d_w), jnp.float32),
        pltpu.VMEM((chunk, d_w), jnp.float32),
        pltpu.VMEM_SHARED((n_rows, d_w), jnp.float32),
        pltpu.SemaphoreType.DMA,
        pltpu.SemaphoreType.DMA,
        pltpu.SemaphoreType.DMA,
        pltpu.SemaphoreType.DMA,
    ]
    if stage:
        scratch.append(pltpu.VMEM_SHARED((n_rows, d_w), jnp.float32))

    @functools.partial(
        pl.kernel,
        mesh=mesh,
        out_type=jax.ShapeDtypeStruct((NC, n_rows, d_w), jnp.float32),
        scratch_types=scratch,
        compiler_params=pltpu.CompilerParams(use_tc_tiling_on_sc=False),
    )
    def seg_kernel(table_hbm, src_hbm, dst_hbm, out_hbm,
                   src_v, dst_v, rows_a, rows_b, acc_sh, sga, sgb, ssa, ssb,
                   *opt):
        c = lax.axis_index("c")
        s = lax.axis_index("s")
        wid = s * NC + c
        tile_base = pl.multiple_of(s * rows_per_tile, 8)

        def _zrow(r, _):
            for cc in range(d_w // 16):
                rows_a[r, pl.ds(cc * 16, 16)] = jnp.zeros((16,), jnp.float32)
            return 0
        lax.fori_loop(0, chunk, _zrow, 0)

        done = 0
        while done < rows_per_tile:
            nr = min(chunk, rows_per_tile - done)
            lo = pl.multiple_of(tile_base + done, 8)
            pltpu.sync_copy(rows_a.at[pl.ds(0, nr)], acc_sh.at[pl.ds(lo, nr)])
            if stage:
                pltpu.sync_copy(table_hbm.at[pl.ds(lo, nr)], rows_b.at[pl.ds(0, nr)])
                pltpu.sync_copy(rows_b.at[pl.ds(0, nr)], opt[0].at[pl.ds(lo, nr)])
            done += nr
        plsc.subcore_barrier()
        table = opt[0] if stage else table_hbm

        def _block(b, _):
            pltpu.sync_copy(src_hbm.at[wid, pl.ds(b * nblk, nblk)], src_v)
            pltpu.sync_copy(dst_hbm.at[wid, pl.ds(b * nblk, nblk)], dst_v)

            def _pair(j2, _u):
                j0 = 2 * j2
                ga = pltpu.async_copy(table.at[src_v.at[j0]], rows_a, sga)
                gb = pltpu.async_copy(table.at[src_v.at[j0 + 1]], rows_b, sgb)
                ga.wait()
                sa = pltpu.async_copy(rows_a, acc_sh.at[dst_v.at[j0]], ssa, add=True)
                gb.wait()
                sb = pltpu.async_copy(rows_b, acc_sh.at[dst_v.at[j0 + 1]], ssb, add=True)
                sa.wait()
                sb.wait()
                return 0
            lax.fori_loop(0, nblk // 2, _pair, 0)
            return 0
        lax.fori_loop(0, npc // nblk, _block, 0)
        plsc.subcore_barrier()

        done = 0
        while done < rows_per_tile:
            nr = min(chunk, rows_per_tile - done)
            lo = pl.multiple_of(tile_base + done, 8)
            pltpu.sync_copy(acc_sh.at[pl.ds(lo, nr)], rows_a.at[pl.ds(0, nr)])
            pltpu.sync_copy(rows_a.at[pl.ds(0, nr)], out_hbm.at[c, pl.ds(lo, nr)])
            done += nr

    return seg_kernel


def _deg_select(agg, idx, w_ref, b_ref):
    acc = jnp.zeros((agg.shape[0], w_ref.shape[2]), jnp.float32)
    for d in range(MAX_DEG):
        y = jnp.dot(agg, w_ref[d], preferred_element_type=jnp.float32) + b_ref[d][None, :]
        acc = jnp.where((idx == d)[:, None], y, acc)
    return acc


def _nf_layer_body(deg_col, h_ref, parts_ref, w_ref, b_ref, out_ref):
    din = h_ref.shape[1]
    agg = h_ref[...] + parts_ref[0, :, :din] + parts_ref[1, :, :din]
    degf = parts_ref[0, :, deg_col] + parts_ref[1, :, deg_col]
    idx = jnp.clip(degf.astype(jnp.int32), 1, MAX_DEG) - 1
    out_ref[...] = jnp.maximum(_deg_select(agg, idx, w_ref, b_ref), 0.0)


def _nf_layer_tc(h, parts, deg_col, w, b, blk):
    n, din = h.shape
    dout = w.shape[2]
    dw = parts.shape[2]
    grid = (n + blk - 1) // blk
    return pl.pallas_call(
        functools.partial(_nf_layer_body, deg_col),
        grid=(grid,),
        in_specs=[
            pl.BlockSpec((blk, din), lambda i: (i, 0)),
            pl.BlockSpec((2, blk, dw), lambda i: (0, i, 0)),
            pl.BlockSpec(w.shape, lambda i: (0, 0, 0)),
            pl.BlockSpec(b.shape, lambda i: (0, 0)),
        ],
        out_specs=pl.BlockSpec((blk, dout), lambda i: (i, 0)),
        out_shape=jax.ShapeDtypeStruct((n, dout), jnp.float32),
    )(h, parts, w, b)


def _readout_body(n_valid, nblocks, deg_col, h_ref, parts_ref, degp_ref, w_ref,
                  b_ref, wn_ref, bn_ref, wt_ref, bt_ref, out_ref, gsum, gmax):
    i = pl.program_id(0)
    blk = h_ref.shape[0]

    agg = h_ref[...] + parts_ref[0] + parts_ref[1]
    degf = degp_ref[0, :, deg_col] + degp_ref[1, :, deg_col]
    idx = jnp.clip(degf.astype(jnp.int32), 1, MAX_DEG) - 1
    h2 = jnp.maximum(_deg_select(agg, idx, w_ref, b_ref), 0.0)

    nf = jnp.dot(h2, wn_ref[...], preferred_element_type=jnp.float32) + bn_ref[...]
    row = i * blk + lax.broadcasted_iota(jnp.int32, (blk, 1), 0)
    valid = row < n_valid
    nsum = jnp.sum(jnp.where(valid, nf, 0.0), axis=0, keepdims=True)
    nmax = jnp.max(jnp.where(valid, nf, -jnp.inf), axis=0, keepdims=True)

    @pl.when(i == 0)
    def _():
        gsum[...] = jnp.zeros_like(gsum)
        gmax[...] = jnp.full_like(gmax, -jnp.inf)

    gsum[...] += nsum
    gmax[...] = jnp.maximum(gmax[...], nmax)

    @pl.when(i == nblocks - 1)
    def _():
        gf = jnp.maximum(jnp.concatenate([gsum[...], gmax[...]], axis=1), 0.0)
        out_ref[...] = jnp.dot(gf, wt_ref[...], preferred_element_type=jnp.float32) + bt_ref[...]


def _readout_tc(h, parts, degparts, deg_col, w, b, wn, bn, wt, bt, blk):
    n, din = h.shape
    p = wn.shape[1]
    out = wt.shape[1]
    dw = degparts.shape[2]
    grid = (n + blk - 1) // blk
    return pl.pallas_call(
        functools.partial(_readout_body, n, grid, deg_col),
        grid=(grid,),
        in_specs=[
            pl.BlockSpec((blk, din), lambda i: (i, 0)),
            pl.BlockSpec((2, blk, din), lambda i: (0, i, 0)),
            pl.BlockSpec((2, blk, dw), lambda i: (0, i, 0)),
            pl.BlockSpec(w.shape, lambda i: (0, 0, 0)),
            pl.BlockSpec(b.shape, lambda i: (0, 0)),
            pl.BlockSpec(wn.shape, lambda i: (0, 0)),
            pl.BlockSpec((1, p), lambda i: (0, 0)),
            pl.BlockSpec(wt.shape, lambda i: (0, 0)),
            pl.BlockSpec((1, out), lambda i: (0, 0)),
        ],
        out_specs=pl.BlockSpec((1, out), lambda i: (0, 0)),
        out_shape=jax.ShapeDtypeStruct((1, out), jnp.float32),
        scratch_shapes=[pltpu.VMEM((1, p), jnp.float32),
                        pltpu.VMEM((1, p), jnp.float32)],
    )(h, parts, degparts, w, b, wn, bn, wt, bt)


def kernel(x, edge_index, W1, b1, W2, b2, W_ntg, b_ntg, W_t, b_t):
    n, d = x.shape
    e = edge_index.shape[1]
    h1dim = W1.shape[2]

    d_aug = ((d + 1 + 15) // 16) * 16
    n_rows = ((n + 1 + 127) // 128) * 128

    ch1, nblk1 = 64, 32
    npc1 = -(-e // (NW * ch1))
    npc1 += (-npc1) % nblk1
    ch2, nblk2 = 128, 40
    npc2 = -(-e // (NW * ch2))
    npc2 += (-npc2) % nblk2
    e_pad = NW * ch1 * npc1
    assert e_pad == NW * ch2 * npc2

    src = edge_index[0].astype(jnp.int32)
    dst = edge_index[1].astype(jnp.int32)
    pad = e_pad - e
    pad_dst = n + jnp.arange(pad, dtype=jnp.int32) % (n_rows - n)
    src_p = jnp.concatenate([src, jnp.zeros((pad,), jnp.int32)])
    dst_p = jnp.concatenate([dst, pad_dst])

    x_aug = jnp.concatenate(
        [x, jnp.ones((n, 1), jnp.float32), jnp.zeros((n, d_aug - d - 1), jnp.float32)],
        axis=1)

    seg1 = _make_segsum(n, d_aug, n_rows, ch1, npc1, nblk1, stage=False)
    parts1 = seg1(x_aug, src_p.reshape(NW, npc1, ch1),
                  dst_p.reshape(NW, npc1, ch1))

    blk = 1024
    h1 = _nf_layer_tc(x, parts1, d, W1, b1, blk)

    h1p = jnp.concatenate(
        [h1, jnp.zeros((n_rows - n, h1dim), jnp.float32)], axis=0)
    seg2 = _make_segsum(n, h1dim, n_rows, ch2, npc2, nblk2, stage=True)
    parts2 = seg2(h1p, src_p.reshape(NW, npc2, ch2),
                  dst_p.reshape(NW, npc2, ch2))

    return _readout_tc(h1, parts2, parts1, d, W2, b2, W_ntg,
                       b_ntg.reshape(1, -1), W_t, b_t.reshape(1, -1), blk)


# speedup vs baseline: 3.5240x; 1.8210x over previous
import functools

import jax
import jax.numpy as jnp
from jax import lax
from jax.experimental import pallas as pl
from jax.experimental.pallas import tpu as pltpu
from jax.experimental.pallas import tpu_sc as plsc

MAX_DEG = 10
NC, NS = 2, 16
NW = NC * NS
CHUNK = 128


def _make_segsum(n_table, d_w, n_rows, chunk, npc, nblk, stage):
    assert d_w % 16 == 0 and n_rows % (NS * 8) == 0
    assert npc % nblk == 0 and nblk % 2 == 0
    rows_per_tile = n_rows // NS
    mesh = plsc.VectorSubcoreMesh(core_axis_name="c", subcore_axis_name="s")

    scratch = [
        pltpu.VMEM((nblk, chunk), jnp.int32),
        pltpu.VMEM((nblk, chunk), jnp.int32),
        pltpu.VMEM((chunk, d_w), jnp.float32),
        pltpu.VMEM((chunk, d_w), jnp.float32),
        pltpu.VMEM_SHARED((n_rows, d_w), jnp.float32),
        pltpu.SemaphoreType.DMA,
        pltpu.SemaphoreType.DMA,
        pltpu.SemaphoreType.DMA,
        pltpu.SemaphoreType.DMA,
    ]
    if stage:
        scratch.append(pltpu.VMEM_SHARED((n_rows, d_w), jnp.float32))

    @functools.partial(
        pl.kernel,
        mesh=mesh,
        out_type=jax.ShapeDtypeStruct((NC, n_rows, d_w), jnp.float32),
        scratch_types=scratch,
        compiler_params=pltpu.CompilerParams(use_tc_tiling_on_sc=False),
    )
    def seg_kernel(table_hbm, src_hbm, dst_hbm, out_hbm,
                   src_v, dst_v, rows_a, rows_b, acc_sh, sga, sgb, ssa, ssb,
                   *opt):
        c = lax.axis_index("c")
        s = lax.axis_index("s")
        wid = s * NC + c
        tile_base = pl.multiple_of(s * rows_per_tile, 8)

        def _zrow(r, _):
            for cc in range(d_w // 16):
                rows_a[r, pl.ds(cc * 16, 16)] = jnp.zeros((16,), jnp.float32)
            return 0
        lax.fori_loop(0, chunk, _zrow, 0)

        done = 0
        while done < rows_per_tile:
            nr = min(chunk, rows_per_tile - done)
            lo = pl.multiple_of(tile_base + done, 8)
            pltpu.sync_copy(rows_a.at[pl.ds(0, nr)], acc_sh.at[pl.ds(lo, nr)])
            if stage:
                pltpu.sync_copy(table_hbm.at[pl.ds(lo, nr)], rows_b.at[pl.ds(0, nr)])
                pltpu.sync_copy(rows_b.at[pl.ds(0, nr)], opt[0].at[pl.ds(lo, nr)])
            done += nr
        plsc.subcore_barrier()
        table = opt[0] if stage else table_hbm

        def _block(b, _):
            pltpu.sync_copy(src_hbm.at[wid, pl.ds(b * nblk, nblk)], src_v)
            pltpu.sync_copy(dst_hbm.at[wid, pl.ds(b * nblk, nblk)], dst_v)

            def _pair(j2, _u):
                j0 = 2 * j2
                ga = pltpu.async_copy(table.at[src_v.at[j0]], rows_a, sga)
                gb = pltpu.async_copy(table.at[src_v.at[j0 + 1]], rows_b, sgb)
                ga.wait()
                sa = pltpu.async_copy(rows_a, acc_sh.at[dst_v.at[j0]], ssa, add=True)
                gb.wait()
                sb = pltpu.async_copy(rows_b, acc_sh.at[dst_v.at[j0 + 1]], ssb, add=True)
                sa.wait()
                sb.wait()
                return 0
            lax.fori_loop(0, nblk // 2, _pair, 0)
            return 0
        lax.fori_loop(0, npc // nblk, _block, 0)
        plsc.subcore_barrier()

        done = 0
        while done < rows_per_tile:
            nr = min(chunk, rows_per_tile - done)
            lo = pl.multiple_of(tile_base + done, 8)
            pltpu.sync_copy(acc_sh.at[pl.ds(lo, nr)], rows_a.at[pl.ds(0, nr)])
            pltpu.sync_copy(rows_a.at[pl.ds(0, nr)], out_hbm.at[c, pl.ds(lo, nr)])
            done += nr

    return seg_kernel


def _deg_select(agg, idx, w_ref, b_ref):
    acc = jnp.zeros((agg.shape[0], w_ref.shape[2]), jnp.float32)
    for d in range(MAX_DEG):
        y = jnp.dot(agg, w_ref[d], preferred_element_type=jnp.float32) + b_ref[d][None, :]
        acc = jnp.where((idx == d)[:, None], y, acc)
    return acc


def _nf_layer_body(deg_col, h_ref, pa_ref, pb_ref, w_ref, b_ref, out_ref):
    din = h_ref.shape[1]
    da = pa_ref.shape[2]
    pa = pa_ref[0] + pa_ref[1]
    pb = pb_ref[0] + pb_ref[1]
    agg = h_ref[...] + jnp.concatenate([pa, pb[:, :din - da]], axis=1)
    degf = pb[:, deg_col]
    idx = jnp.clip(degf.astype(jnp.int32), 1, MAX_DEG) - 1
    out_ref[...] = jnp.maximum(_deg_select(agg, idx, w_ref, b_ref), 0.0)


def _nf_layer_tc(h, parts_a, parts_b, deg_col, w, b, blk):
    n, din = h.shape
    dout = w.shape[2]
    da = parts_a.shape[2]
    db = parts_b.shape[2]
    grid = (n + blk - 1) // blk
    return pl.pallas_call(
        functools.partial(_nf_layer_body, deg_col),
        grid=(grid,),
        in_specs=[
            pl.BlockSpec((blk, din), lambda i: (i, 0)),
            pl.BlockSpec((2, blk, da), lambda i: (0, i, 0)),
            pl.BlockSpec((2, blk, db), lambda i: (0, i, 0)),
            pl.BlockSpec(w.shape, lambda i: (0, 0, 0)),
            pl.BlockSpec(b.shape, lambda i: (0, 0)),
        ],
        out_specs=pl.BlockSpec((blk, dout), lambda i: (i, 0)),
        out_shape=jax.ShapeDtypeStruct((n, dout), jnp.float32),
    )(h, parts_a, parts_b, w, b)


def _readout_body(n_valid, nblocks, deg_col, h_ref, parts_ref, degp_ref, w_ref,
                  b_ref, wn_ref, bn_ref, wt_ref, bt_ref, out_ref, gsum, gmax):
    i = pl.program_id(0)
    blk = h_ref.shape[0]

    agg = h_ref[...] + parts_ref[0] + parts_ref[1]
    degf = degp_ref[0, :, deg_col] + degp_ref[1, :, deg_col]
    idx = jnp.clip(degf.astype(jnp.int32), 1, MAX_DEG) - 1
    h2 = jnp.maximum(_deg_select(agg, idx, w_ref, b_ref), 0.0)

    nf = jnp.dot(h2, wn_ref[...], preferred_element_type=jnp.float32) + bn_ref[...]
    row = i * blk + lax.broadcasted_iota(jnp.int32, (blk, 1), 0)
    valid = row < n_valid
    nsum = jnp.sum(jnp.where(valid, nf, 0.0), axis=0, keepdims=True)
    nmax = jnp.max(jnp.where(valid, nf, -jnp.inf), axis=0, keepdims=True)

    @pl.when(i == 0)
    def _():
        gsum[...] = jnp.zeros_like(gsum)
        gmax[...] = jnp.full_like(gmax, -jnp.inf)

    gsum[...] += nsum
    gmax[...] = jnp.maximum(gmax[...], nmax)

    @pl.when(i == nblocks - 1)
    def _():
        gf = jnp.maximum(jnp.concatenate([gsum[...], gmax[...]], axis=1), 0.0)
        out_ref[...] = jnp.dot(gf, wt_ref[...], preferred_element_type=jnp.float32) + bt_ref[...]


def _readout_tc(h, parts, degparts, deg_col, w, b, wn, bn, wt, bt, blk):
    n, din = h.shape
    p = wn.shape[1]
    out = wt.shape[1]
    dw = degparts.shape[2]
    grid = (n + blk - 1) // blk
    return pl.pallas_call(
        functools.partial(_readout_body, n, grid, deg_col),
        grid=(grid,),
        in_specs=[
            pl.BlockSpec((blk, din), lambda i: (i, 0)),
            pl.BlockSpec((2, blk, din), lambda i: (0, i, 0)),
            pl.BlockSpec((2, blk, dw), lambda i: (0, i, 0)),
            pl.BlockSpec(w.shape, lambda i: (0, 0, 0)),
            pl.BlockSpec(b.shape, lambda i: (0, 0)),
            pl.BlockSpec(wn.shape, lambda i: (0, 0)),
            pl.BlockSpec((1, p), lambda i: (0, 0)),
            pl.BlockSpec(wt.shape, lambda i: (0, 0)),
            pl.BlockSpec((1, out), lambda i: (0, 0)),
        ],
        out_specs=pl.BlockSpec((1, out), lambda i: (0, 0)),
        out_shape=jax.ShapeDtypeStruct((1, out), jnp.float32),
        scratch_shapes=[pltpu.VMEM((1, p), jnp.float32),
                        pltpu.VMEM((1, p), jnp.float32)],
    )(h, parts, degparts, w, b, wn, bn, wt, bt)


def kernel(x, edge_index, W1, b1, W2, b2, W_ntg, b_ntg, W_t, b_t):
    n, d = x.shape
    e = edge_index.shape[1]
    h1dim = W1.shape[2]

    n_rows = ((n + 1 + 127) // 128) * 128

    da = 80
    db = ((d - da + 1 + 15) // 16) * 16
    deg_col = d - da

    ch1, nblk1 = 128, 20
    npc1 = -(-e // (NW * ch1))
    npc1 += (-npc1) % nblk1
    ch2, nblk2 = 128, 20
    npc2 = -(-e // (NW * ch2))
    npc2 += (-npc2) % nblk2
    e_pad = NW * ch1 * npc1
    assert e_pad == NW * ch2 * npc2

    src = edge_index[0].astype(jnp.int32)
    dst = edge_index[1].astype(jnp.int32)
    pad = e_pad - e
    pad_dst = n + jnp.arange(pad, dtype=jnp.int32) % (n_rows - n)
    src_p = jnp.concatenate([src, jnp.zeros((pad,), jnp.int32)])
    dst_p = jnp.concatenate([dst, pad_dst])

    rpad = jnp.zeros((n_rows - n, 1), jnp.float32)
    xa = jnp.concatenate([x[:, :da],
                          jnp.broadcast_to(rpad, (n_rows - n, da))], axis=0)
    xb_cols = jnp.concatenate(
        [x[:, da:], jnp.ones((n, 1), jnp.float32),
         jnp.zeros((n, db - (d - da) - 1), jnp.float32)], axis=1)
    xb = jnp.concatenate([xb_cols,
                          jnp.broadcast_to(rpad, (n_rows - n, db))], axis=0)

    src3 = src_p.reshape(NW, npc1, ch1)
    dst3 = dst_p.reshape(NW, npc1, ch1)
    seg1a = _make_segsum(n, da, n_rows, ch1, npc1, nblk1, stage=True)
    seg1b = _make_segsum(n, db, n_rows, ch1, npc1, nblk1, stage=True)
    parts1a = seg1a(xa, src3, dst3)
    parts1b = seg1b(xb, src3, dst3)

    blk = 1024
    h1 = _nf_layer_tc(x, parts1a, parts1b, deg_col, W1, b1, blk)

    h1p = jnp.concatenate(
        [h1, jnp.zeros((n_rows - n, h1dim), jnp.float32)], axis=0)
    seg2 = _make_segsum(n, h1dim, n_rows, ch2, npc2, nblk2, stage=True)
    parts2 = seg2(h1p, src3, dst3)

    return _readout_tc(h1, parts2, parts1b, deg_col, W2, b2, W_ntg,
                       b_ntg.reshape(1, -1), W_t, b_t.reshape(1, -1), blk)
